# Initial kernel scaffold; baseline (speedup 1.0000x reference)
import jax
import jax.numpy as jnp
from jax.experimental import pallas as pl


def kernel(x, edge_index, edge_attr, batch, W1, We, W2, b2, Wc, bc):
    raise NotImplementedError("write your pallas kernel here")


# speedup vs baseline: 4.5560x; 4.5560x over previous
import functools

import jax
import jax.numpy as jnp
import numpy as np
from jax import lax
from jax.experimental import pallas as pl
from jax.experimental.pallas import tpu as pltpu
from jax.experimental.pallas import tpu_sc as plsc

N = 10000
E = 320000
D_FEAT = 128
D_GLOBAL = 16
D_EDGE = 4
D_EMB = 128
N_CLASSES = 8
SIGMA2 = 1.0 + 1e-06
_I0 = np.int32(0)

BE = 2000
NB_A = E // BE
WREP = 16


def _edge_w_body(a_ref, we_ref, out_ref, mn_ref):
    ph = pl.program_id(0)
    j = pl.program_id(1)
    blk = a_ref[...]

    @pl.when(jnp.logical_and(ph == 0, j == 0))
    def _():
        mn_ref[...] = jnp.full((1, D_EDGE), jnp.inf, jnp.float32)

    @pl.when(ph == 0)
    def _():
        mn_ref[...] = jnp.minimum(mn_ref[...],
                                  jnp.min(blk, axis=0, keepdims=True))
        out_ref[...] = jnp.zeros_like(out_ref)

    @pl.when(ph == 1)
    def _():
        p = blk - mn_ref[...]
        ex = jnp.exp(-(p * p) / SIGMA2)
        s = jnp.sum(ex * we_ref[...], axis=1, keepdims=True)
        w = 1.0 / (1.0 + jnp.exp(-s))
        out_ref[...] = jnp.broadcast_to(w, (BE, WREP))


def _edge_weights(edge_attr, We_row):
    return pl.pallas_call(
        _edge_w_body,
        grid=(2, NB_A),
        in_specs=[
            pl.BlockSpec((BE, D_EDGE), lambda ph, j: (j, _I0)),
            pl.BlockSpec((1, D_EDGE), lambda ph, j: (_I0, _I0)),
        ],
        out_specs=pl.BlockSpec((BE, WREP), lambda ph, j: (j, _I0)),
        out_shape=jax.ShapeDtypeStruct((E, WREP), jnp.float32),
        name="edge_w",
        scratch_shapes=[pltpu.VMEM((1, D_EDGE), jnp.float32)],
    )(edge_attr, We_row)


BN = 1000
NB_H = N // BN


def _h_body(x_ref, w1_ref, out_ref):
    out_ref[...] = jnp.dot(x_ref[...], w1_ref[...],
                           preferred_element_type=jnp.float32)


def _node_emb(x_real, W1):
    return pl.pallas_call(
        _h_body,
        grid=(NB_H,),
        in_specs=[
            pl.BlockSpec((BN, D_FEAT), lambda j: (j, _I0)),
            pl.BlockSpec((D_FEAT, D_EMB), lambda j: (_I0, _I0)),
        ],
        out_specs=pl.BlockSpec((BN, D_EMB), lambda j: (j, _I0)),
        out_shape=jax.ShapeDtypeStruct((N, D_EMB), jnp.float32),
        name="h_mm",
    )(x_real, W1)


NTILES = 32
EPT = E // NTILES
C = 80
K = EPT // C
G = 25
NG = K // G
RPT = N // 16


def _sc_body(h_hbm, src_hbm, dst_hbm, w_hbm, out_hbm,
             src_v, dst_v, w_v, rows_v, agg_sh, sem):
    cid = lax.axis_index("c")
    sid = lax.axis_index("s")
    wid = cid * 16 + sid

    def _zb(i, carry):
        for k in range(8):
            rows_v[i, pl.ds(k * 16, 16)] = jnp.zeros((16,), jnp.float32)
        return carry
    lax.fori_loop(jnp.int32(0), jnp.int32(C), _zb, jnp.int32(0))
    for t in range(7):
        pltpu.sync_copy(rows_v, agg_sh.at[pl.ds(sid * RPT + t * C, C)])
    pltpu.sync_copy(rows_v.at[pl.ds(0, RPT - 7 * C)],
                    agg_sh.at[pl.ds(sid * RPT + 7 * C, RPT - 7 * C)])
    plsc.subcore_barrier()

    for g in range(NG):
        pltpu.sync_copy(src_hbm.at[wid, jnp.int32(g)], src_v)
        pltpu.sync_copy(dst_hbm.at[wid, jnp.int32(g)], dst_v)

        def _chunk(j, carry):
            pltpu.sync_copy(w_hbm.at[wid, jnp.int32(g * G) + j], w_v)
            pltpu.async_copy(h_hbm.at[src_v.at[j]], rows_v, sem).wait()

            def _scale(e, c2):
                wj = w_v[e, :]
                for k in range(8):
                    sl = pl.ds(k * 16, 16)
                    rows_v[e, sl] = rows_v[e, sl] * wj
                return c2
            lax.fori_loop(jnp.int32(0), jnp.int32(C), _scale, jnp.int32(0))

            pltpu.sync_copy(rows_v, agg_sh.at[dst_v.at[j]], add=True)
            return carry
        lax.fori_loop(jnp.int32(0), jnp.int32(G), _chunk, jnp.int32(0))
    plsc.subcore_barrier()

    pltpu.sync_copy(agg_sh.at[pl.ds(sid * RPT, RPT)], out_hbm.at[wid])


def _sc_scatter(h, src3d, dst3d, w4d):
    mesh = plsc.VectorSubcoreMesh(core_axis_name="c", subcore_axis_name="s")
    fn = functools.partial(
        pl.kernel,
        out_type=jax.ShapeDtypeStruct((NTILES, RPT, D_EMB), jnp.float32),
        mesh=mesh,
        name="sc_scatter",
        scratch_types=[
            pltpu.VMEM((G, C), jnp.int32),
            pltpu.VMEM((G, C), jnp.int32),
            pltpu.VMEM((C, WREP), jnp.float32),
            pltpu.VMEM((C, D_EMB), jnp.float32),
            pltpu.VMEM_SHARED((N, D_EMB), jnp.float32),
            pltpu.SemaphoreType.DMA,
        ],
    )(_sc_body)
    return fn(h, src3d, dst3d, w4d)


def _final_body(p0_ref, p1_ref, w2_ref, b2_ref, wc_ref, bc_ref, g_ref,
                out_ref, acc_ref):
    j = pl.program_id(0)

    @pl.when(j == 0)
    def _():
        acc_ref[...] = jnp.zeros_like(acc_ref)
        out_ref[...] = jnp.zeros_like(out_ref)

    s = p0_ref[...] + p1_ref[...]
    t = jnp.dot(s, w2_ref[...], preferred_element_type=jnp.float32)
    t = jnp.maximum(t + b2_ref[...], 0.0)
    acc_ref[...] += jnp.sum(t, axis=0, keepdims=True)

    @pl.when(j == NB_H - 1)
    def _():
        o = jnp.dot(acc_ref[...], wc_ref[pl.ds(0, D_EMB), :],
                    preferred_element_type=jnp.float32)
        o += jnp.dot(g_ref[...], wc_ref[pl.ds(D_EMB, D_GLOBAL), :],
                     preferred_element_type=jnp.float32)
        out_ref[...] = o + bc_ref[...]


def _final(parts, W2, b2r, Wc, bcr, g):
    return pl.pallas_call(
        _final_body,
        grid=(NB_H,),
        in_specs=[
            pl.BlockSpec((BN, D_EMB), lambda j: (j, _I0)),
            pl.BlockSpec((BN, D_EMB), lambda j: (j + NB_H, _I0)),
            pl.BlockSpec((D_EMB, D_EMB), lambda j: (_I0, _I0)),
            pl.BlockSpec((1, D_EMB), lambda j: (_I0, _I0)),
            pl.BlockSpec((D_EMB + D_GLOBAL, N_CLASSES), lambda j: (_I0, _I0)),
            pl.BlockSpec((1, N_CLASSES), lambda j: (_I0, _I0)),
            pl.BlockSpec((1, D_GLOBAL), lambda j: (_I0, _I0)),
        ],
        out_specs=pl.BlockSpec((1, N_CLASSES), lambda j: (_I0, _I0)),
        out_shape=jax.ShapeDtypeStruct((1, N_CLASSES), jnp.float32),
        name="final",
        scratch_shapes=[pltpu.VMEM((1, D_EMB), jnp.float32)],
    )(parts, parts, W2, b2r, Wc, bcr, g)


def kernel(x, edge_index, edge_attr, batch, W1, We, W2, b2, Wc, bc):
    src = edge_index[0].astype(jnp.int32)
    dst = edge_index[1].astype(jnp.int32)
    src3d = src.reshape(NTILES, NG, G, C)
    dst3d = dst.reshape(NTILES, NG, G, C)
    x_real = x[:, :D_FEAT]
    g = x[0:1, D_FEAT:]

    w_e = _edge_weights(edge_attr, We.reshape(1, D_EDGE))
    w4d = w_e.reshape(NTILES, K, C, WREP)
    h = _node_emb(x_real, W1)
    parts = _sc_scatter(h, src3d, dst3d, w4d)
    parts = parts.reshape(2 * N, D_EMB)
    out = _final(parts, W2, b2.reshape(1, D_EMB), Wc,
                 bc.reshape(1, N_CLASSES), g)
    return out
